# baseline (speedup 1.0000x reference)
import functools

import jax
import jax.numpy as jnp
from jax import lax
from jax.experimental import pallas as pl
from jax.experimental.pallas import tpu as pltpu
from jax.experimental.pallas import tpu_sc as plsc

_NC = 2
_NS = 16
_NW = _NC * _NS
_L = 16
_CHUNK = 16


@jax.jit
def _sc_embed(x, table, pe):
    b, s = x.shape
    d = table.shape[1]
    s_per_w = s // _NW
    pe_half = s_per_w // 2
    n_super = s_per_w // _CHUNK
    mesh = plsc.VectorSubcoreMesh(core_axis_name="c", subcore_axis_name="s")

    @functools.partial(
        pl.kernel,
        mesh=mesh,
        out_type=jax.ShapeDtypeStruct((b, s, d), jnp.float32),
        scratch_types=[
            pltpu.VMEM((b * s_per_w,), jnp.int32),
            pltpu.VMEM((pe_half, d), jnp.float32),
            pltpu.VMEM((_CHUNK, d), jnp.float32),
            pltpu.VMEM((_CHUNK, d), jnp.float32),
            pltpu.VMEM((_CHUNK, d), jnp.float32),
            pltpu.VMEM((_CHUNK, d), jnp.float32),
            pltpu.SemaphoreType.DMA,
            pltpu.SemaphoreType.DMA,
            pltpu.SemaphoreType.DMA,
            pltpu.SemaphoreType.DMA,
            pltpu.SemaphoreType.DMA,
            pltpu.SemaphoreType.DMA,
            pltpu.SemaphoreType.DMA,
            pltpu.SemaphoreType.DMA,
            pltpu.SemaphoreType.DMA,
            pltpu.SemaphoreType.DMA,
        ],
    )
    def k(x_hbm, tab_hbm, pe_hbm, out_hbm,
          idx_v, pe_v, r0, r1, r2, r3,
          g0, g1, g2, g3, o0, o1, o2, o3, psem, isem):
        rows = (r0, r1, r2, r3)
        gs = (g0, g1, g2, g3)
        os = (o0, o1, o2, o3)

        wid = lax.axis_index("s") * _NC + lax.axis_index("c")
        s_base = wid * s_per_w

        pedesc = pltpu.async_copy(
            pe_hbm.at[0, pl.ds(s_base, pe_half)], pe_v, psem)
        idescs = [
            pltpu.async_copy(x_hbm.at[bi, pl.ds(s_base, s_per_w)],
                             idx_v.at[pl.ds(bi * s_per_w, s_per_w)], isem)
            for bi in range(b)
        ]
        for dsc in idescs:
            dsc.wait()

        def fire_gather(tn, pn):
            pltpu.async_copy(
                tab_hbm.at[idx_v.at[pl.ds(pn * s_per_w + tn * _CHUNK,
                                          _CHUNK)]],
                rows[pn], gs[pn])

        def wait_gather(p):
            pltpu.make_async_copy(
                pe_hbm.at[0, pl.ds(0, _CHUNK)], rows[p], gs[p]).wait()

        def fire_store(t, p):
            pltpu.async_copy(
                rows[p], out_hbm.at[p, pl.ds(s_base + t * _CHUNK, _CHUNK)],
                os[p])

        def wait_store(q):
            pltpu.make_async_copy(
                rows[q], out_hbm.at[0, pl.ds(0, _CHUNK)], os[q]).wait()

        def add_pe(t, p):
            buf = rows[p]
            pr0 = (t % (pe_half // _CHUNK)) * _CHUNK

            def body_r(r, carry):
                pr = pr0 + r

                def body_j(j, carry2):
                    for kk in range(16):
                        sl = pl.ds(j * 256 + kk * _L, _L)
                        plsc.addupdate(buf.at[r, sl], pe_v[pr, sl])
                    return carry2

                lax.fori_loop(0, d // _L // 16, body_j, 0)
                return carry

            lax.fori_loop(0, _CHUNK, body_r, 0)

        for p in range(3):
            fire_gather(0, p)
        pedesc.wait()

        def iteration(t, p, drain_store, fire_ahead):
            wait_gather(p)
            add_pe(t, p)
            fire_store(t, p)
            if fire_ahead:
                if drain_store:
                    wait_store((p + 3) % b)
                tn = t + (p + 3) // b
                fire_gather(tn, (p + 3) % b)

        iteration(0, 0, False, True)
        for p in range(1, b):
            iteration(0, p, True, True)

        def body_t(t, carry):
            wait_gather(0)

            @pl.when(t == n_super // 2)
            def _():
                pltpu.sync_copy(
                    pe_hbm.at[0, pl.ds(s_base + pe_half, pe_half)], pe_v)

            add_pe(t, 0)
            fire_store(t, 0)
            wait_store(3)
            fire_gather(t, 3)
            for p in range(1, b):
                iteration(t, p, True, True)
            return carry

        lax.fori_loop(1, n_super - 1, body_t, 0)

        iteration(n_super - 1, 0, True, True)
        for p in range(1, b):
            iteration(n_super - 1, p, False, False)

        for q in range(b):
            wait_store(q)

    return k(x, table, pe)


def kernel(x, table, pe):
    return _sc_embed(x.astype(jnp.int32), table, pe)


# speedup vs baseline: 1.5442x; 1.5442x over previous
import functools

import jax
import jax.numpy as jnp
from jax import lax
from jax.experimental import pallas as pl
from jax.experimental.pallas import tpu as pltpu
from jax.experimental.pallas import tpu_sc as plsc

_NC = 2
_NS = 16
_NW = _NC * _NS
_L = 16


@functools.partial(jax.jit, static_argnames=("chunk",))
def _sc_embed(x, table, pe, *, chunk):
    b, s = x.shape
    d = table.shape[1]
    s_per_w = s // _NW
    n_chunks = s_per_w // chunk
    n_iter = n_chunks * b
    mesh = plsc.VectorSubcoreMesh(core_axis_name="c", subcore_axis_name="s")

    @functools.partial(
        pl.kernel,
        mesh=mesh,
        out_type=jax.ShapeDtypeStruct((b, s, d), jnp.float32),
        scratch_types=[
            pltpu.VMEM((b * s_per_w,), jnp.int32),
            pltpu.VMEM((chunk, d), jnp.float32),
            pltpu.VMEM((chunk, d), jnp.float32),
            pltpu.VMEM((chunk, d), jnp.float32),
            pltpu.VMEM((chunk, d), jnp.float32),
            pltpu.SemaphoreType.DMA,
            pltpu.SemaphoreType.DMA,
            pltpu.SemaphoreType.DMA,
            pltpu.SemaphoreType.DMA,
            pltpu.SemaphoreType.DMA,
            pltpu.SemaphoreType.DMA,
        ],
    )
    def k(x_hbm, tab_hbm, pe_hbm, out_hbm,
          idx_v, rows0, rows1, pe0, pe1, g0, g1, o0, o1, psem, isem):
        rows = (rows0, rows1)
        pes = (pe0, pe1)
        gsems = (g0, g1)
        osems = (o0, o1)

        wid = lax.axis_index("s") * _NC + lax.axis_index("c")
        s_base = wid * s_per_w

        idescs = [
            pltpu.async_copy(x_hbm.at[bi, pl.ds(s_base, s_per_w)],
                             idx_v.at[pl.ds(bi * s_per_w, s_per_w)], isem)
            for bi in range(b)
        ]
        pedesc0 = pltpu.async_copy(pe_hbm.at[0, pl.ds(s_base, chunk)],
                                   pe0, psem)
        for dsc in idescs:
            dsc.wait()

        def idx_view(c, bi):
            return idx_v.at[pl.ds(bi * s_per_w + c * chunk, chunk)]

        def out_view(c, bi):
            return out_hbm.at[bi, pl.ds(s_base + c * chunk, chunk)]

        gdesc = [None, None]
        odesc = [None, None]
        pedesc = None
        gdesc[0] = pltpu.async_copy(tab_hbm.at[idx_view(0, 0)], rows0, g0)

        for i in range(n_iter):
            p = i % 2
            c, bi = divmod(i, b)
            if bi == 0 and c + 1 < n_chunks:
                pedesc = pltpu.async_copy(
                    pe_hbm.at[0, pl.ds(s_base + (c + 1) * chunk, chunk)],
                    pes[(c + 1) % 2], psem)
            if i == 0:
                pedesc0.wait()
            if bi == 0 and c > 0:
                pedesc.wait()
            gdesc[p].wait()
            if i + 1 < n_iter:
                if i >= 1 and odesc[1 - p] is not None:
                    odesc[1 - p].wait()
                cn, bn = divmod(i + 1, b)
                gdesc[1 - p] = pltpu.async_copy(
                    tab_hbm.at[idx_view(cn, bn)], rows[1 - p], gsems[1 - p])

            pe_buf = pes[c % 2]
            rows_buf = rows[p]

            def add_row(r, carry):
                for j in range(d // _L):
                    sl = pl.ds(j * _L, _L)
                    plsc.addupdate(rows_buf.at[r, sl], pe_buf[r, sl])
                return carry

            lax.fori_loop(0, chunk, add_row, 0)
            odesc[p] = pltpu.async_copy(rows_buf, out_view(c, bi), osems[p])

        odesc[0].wait()
        odesc[1].wait()

    return k(x, table, pe)


def kernel(x, table, pe):
    return _sc_embed(x.astype(jnp.int32), table, pe, chunk=32)


# speedup vs baseline: 1.5460x; 1.0011x over previous
import functools

import jax
import jax.numpy as jnp
from jax import lax
from jax.experimental import pallas as pl
from jax.experimental.pallas import tpu as pltpu
from jax.experimental.pallas import tpu_sc as plsc

_NC = 2
_NS = 16
_NW = _NC * _NS
_L = 16


@functools.partial(jax.jit, static_argnames=("chunk",))
def _sc_embed(x, table, pe, *, chunk):
    b, s = x.shape
    d = table.shape[1]
    s_per_w = s // _NW
    n_chunks = s_per_w // chunk
    n_iter = n_chunks * b
    mesh = plsc.VectorSubcoreMesh(core_axis_name="c", subcore_axis_name="s")

    @functools.partial(
        pl.kernel,
        mesh=mesh,
        out_type=jax.ShapeDtypeStruct((b, s, d), jnp.float32),
        scratch_types=[
            pltpu.VMEM((b * s_per_w,), jnp.int32),
            pltpu.VMEM((chunk, d), jnp.float32),
            pltpu.VMEM((chunk, d), jnp.float32),
            pltpu.VMEM((chunk, d), jnp.float32),
            pltpu.VMEM((chunk, d), jnp.float32),
            pltpu.SemaphoreType.DMA,
            pltpu.SemaphoreType.DMA,
            pltpu.SemaphoreType.DMA,
            pltpu.SemaphoreType.DMA,
            pltpu.SemaphoreType.DMA,
            pltpu.SemaphoreType.DMA,
        ],
    )
    def k(x_hbm, tab_hbm, pe_hbm, out_hbm,
          idx_v, rows0, rows1, pe0, pe1, g0, g1, o0, o1, psem, isem):
        rows = (rows0, rows1)
        pes = (pe0, pe1)
        gsems = (g0, g1)
        osems = (o0, o1)

        wid = lax.axis_index("s") * _NC + lax.axis_index("c")
        s_base = wid * s_per_w

        idescs = [
            pltpu.async_copy(x_hbm.at[bi, pl.ds(s_base, s_per_w)],
                             idx_v.at[pl.ds(bi * s_per_w, s_per_w)], isem)
            for bi in range(b)
        ]
        pedesc0 = pltpu.async_copy(pe_hbm.at[0, pl.ds(s_base, chunk)],
                                   pe0, psem)

        def idx_view(c, bi):
            return idx_v.at[pl.ds(bi * s_per_w + c * chunk, chunk)]

        def out_view(c, bi):
            return out_hbm.at[bi, pl.ds(s_base + c * chunk, chunk)]

        gdesc = [None, None]
        odesc = [None, None]
        pedesc = None
        idescs[0].wait()
        gdesc[0] = pltpu.async_copy(tab_hbm.at[idx_view(0, 0)], rows0, g0)
        for dsc in idescs[1:]:
            dsc.wait()

        for i in range(n_iter):
            p = i % 2
            c, bi = divmod(i, b)
            if bi == 0 and c + 1 < n_chunks:
                pedesc = pltpu.async_copy(
                    pe_hbm.at[0, pl.ds(s_base + (c + 1) * chunk, chunk)],
                    pes[(c + 1) % 2], psem)
            if i == 0:
                pedesc0.wait()
            if bi == 0 and c > 0:
                pedesc.wait()
            gdesc[p].wait()
            if i + 1 < n_iter:
                if i >= 1 and odesc[1 - p] is not None:
                    odesc[1 - p].wait()
                cn, bn = divmod(i + 1, b)
                gdesc[1 - p] = pltpu.async_copy(
                    tab_hbm.at[idx_view(cn, bn)], rows[1 - p], gsems[1 - p])

            pe_buf = pes[c % 2]
            rows_buf = rows[p]

            def add_row(r, carry):
                for j in range(d // _L):
                    sl = pl.ds(j * _L, _L)
                    plsc.addupdate(rows_buf.at[r, sl], pe_buf[r, sl])
                return carry

            lax.fori_loop(0, chunk, add_row, 0)
            odesc[p] = pltpu.async_copy(rows_buf, out_view(c, bi), osems[p])

        odesc[0].wait()
        odesc[1].wait()

    return k(x, table, pe)


def kernel(x, table, pe):
    return _sc_embed(x.astype(jnp.int32), table, pe, chunk=32)


# speedup vs baseline: 1.5559x; 1.0064x over previous
import functools

import jax
import jax.numpy as jnp
from jax import lax
from jax.experimental import pallas as pl
from jax.experimental.pallas import tpu as pltpu
from jax.experimental.pallas import tpu_sc as plsc

_NC = 2
_NS = 16
_NW = _NC * _NS
_L = 16


@functools.partial(jax.jit, static_argnames=("chunk",))
def _sc_embed(x, table, pe, *, chunk):
    b, s = x.shape
    d = table.shape[1]
    s_per_w = s // _NW
    n_chunks = s_per_w // chunk
    n_iter = n_chunks * b
    mesh = plsc.VectorSubcoreMesh(core_axis_name="c", subcore_axis_name="s")

    @functools.partial(
        pl.kernel,
        mesh=mesh,
        out_type=jax.ShapeDtypeStruct((b, s, d), jnp.float32),
        scratch_types=[
            pltpu.VMEM((b * s_per_w,), jnp.int32),
            pltpu.VMEM((chunk, d), jnp.float32),
            pltpu.VMEM((chunk, d), jnp.float32),
            pltpu.VMEM((chunk, d), jnp.float32),
            pltpu.VMEM((chunk, d), jnp.float32),
            pltpu.VMEM((chunk, d), jnp.float32),
            pltpu.SemaphoreType.DMA,
            pltpu.SemaphoreType.DMA,
            pltpu.SemaphoreType.DMA,
            pltpu.SemaphoreType.DMA,
            pltpu.SemaphoreType.DMA,
            pltpu.SemaphoreType.DMA,
            pltpu.SemaphoreType.DMA,
            pltpu.SemaphoreType.DMA,
        ],
    )
    def k(x_hbm, tab_hbm, pe_hbm, out_hbm,
          idx_v, rows0, rows1, rows2, pe0, pe1,
          g0, g1, g2, o0, o1, o2, psem, isem):
        rows = (rows0, rows1, rows2)
        pes = (pe0, pe1)
        gsems = (g0, g1, g2)
        osems = (o0, o1, o2)

        wid = lax.axis_index("s") * _NC + lax.axis_index("c")
        s_base = wid * s_per_w

        idescs = [
            pltpu.async_copy(x_hbm.at[bi, pl.ds(s_base, s_per_w)],
                             idx_v.at[pl.ds(bi * s_per_w, s_per_w)], isem)
            for bi in range(b)
        ]
        pedesc0 = pltpu.async_copy(pe_hbm.at[0, pl.ds(s_base, chunk)],
                                   pe0, psem)

        def idx_view(c, bi):
            return idx_v.at[pl.ds(bi * s_per_w + c * chunk, chunk)]

        def out_view(c, bi):
            return out_hbm.at[bi, pl.ds(s_base + c * chunk, chunk)]

        gdesc = [None, None, None]
        odesc = [None, None, None]
        pedesc = None
        idescs[0].wait()
        gdesc[0] = pltpu.async_copy(tab_hbm.at[idx_view(0, 0)], rows0, g0)
        for dsc in idescs[1:]:
            dsc.wait()
        gdesc[1] = pltpu.async_copy(tab_hbm.at[idx_view(0, 1)], rows1, g1)

        for i in range(n_iter):
            p = i % 3
            c, bi = divmod(i, b)
            if bi == 0 and c + 1 < n_chunks:
                pedesc = pltpu.async_copy(
                    pe_hbm.at[0, pl.ds(s_base + (c + 1) * chunk, chunk)],
                    pes[(c + 1) % 2], psem)
            if i == 0:
                pedesc0.wait()
            if bi == 0 and c > 0:
                pedesc.wait()
            gdesc[p].wait()
            if i + 2 < n_iter:
                q = (i + 2) % 3
                if odesc[q] is not None:
                    odesc[q].wait()
                cn, bn = divmod(i + 2, b)
                gdesc[q] = pltpu.async_copy(
                    tab_hbm.at[idx_view(cn, bn)], rows[q], gsems[q])

            pe_buf = pes[c % 2]
            rows_buf = rows[p]

            def add_row(r, carry):
                for j in range(d // _L):
                    sl = pl.ds(j * _L, _L)
                    plsc.addupdate(rows_buf.at[r, sl], pe_buf[r, sl])
                return carry

            lax.fori_loop(0, chunk, add_row, 0)
            odesc[p] = pltpu.async_copy(rows_buf, out_view(c, bi), osems[p])

        odesc[0].wait()
        odesc[1].wait()
        odesc[2].wait()

    return k(x, table, pe)


def kernel(x, table, pe):
    return _sc_embed(x.astype(jnp.int32), table, pe, chunk=32)
